# Initial kernel scaffold; baseline (speedup 1.0000x reference)
import jax
import jax.numpy as jnp
from jax.experimental import pallas as pl


def kernel(z, x, params):
    raise NotImplementedError("write your pallas kernel here")


# speedup vs baseline: 11.7754x; 11.7754x over previous
import functools
import jax
import jax.numpy as jnp
import numpy as np
from jax import lax
from jax.experimental import pallas as pl
from jax.experimental.pallas import tpu as pltpu
from jax.experimental.pallas import tpu_sc as plsc

DIM = 128
BASIS = 64
INNER = DIM + BASIS
HID = 64
N = 2048
K = 16
NB = 2
TW = 128
RB = 256
RE = 256
BIG = 3.4e38

SC_NC = 2
SC_NS = 16
SC_NW = SC_NC * SC_NS
EDGES = NB * N * K
E_PER_W = EDGES // SC_NW
CHUNK = 512
NCHUNK = E_PER_W // CHUNK


def _silu(t):
    return t * (1.0 / (1.0 + jnp.exp(-t)))


def _knn_prep_body(coord_ref, coordT_ref, h_ref, eW1_ref, eb1_ref,
                   nbr_ref, a_ref, tab_ref):
    b = pl.program_id(0)
    r = pl.program_id(1)
    cb = coord_ref[0]
    cT = coordT_ref[0]
    d2 = jnp.zeros((RB, N), jnp.float32)
    for dd in range(3):
        col = jnp.broadcast_to(cT[dd:dd + 1, :], (RB, N))
        row = jnp.broadcast_to(cb[:, dd:dd + 1], (RB, N))
        diff = row - col
        d2 = d2 + diff * diff
    iota_col = lax.broadcasted_iota(jnp.int32, (RB, N), 1)
    row_glob = r * RB + lax.broadcasted_iota(jnp.int32, (RB, N), 0)
    d2 = jnp.where(iota_col == row_glob, d2 + 1e9, d2)
    cols = []
    for _ in range(K):
        mn = jnp.min(d2, axis=1, keepdims=True)
        idx = jnp.min(jnp.where(d2 <= mn, iota_col, N), axis=1, keepdims=True)
        cols.append(idx)
        d2 = jnp.where(iota_col == idx, BIG, d2)
    nbr = jnp.concatenate(cols, axis=1) + b * N
    nbr_ref[0] = nbr
    hb = h_ref[0]
    W1a = eW1_ref[0:DIM, :]
    W1b = eW1_ref[DIM:2 * DIM, :]
    a_ref[0] = (jnp.dot(hb, W1a, preferred_element_type=jnp.float32)
                + eb1_ref[:][None, :])
    bv = jnp.dot(hb, W1b, preferred_element_type=jnp.float32)
    tab_ref[0] = jnp.concatenate(
        [bv, cb, jnp.zeros((RB, TW - HID - 3), jnp.float32)], axis=1)


def _knn_prep(coord, coordT, h, eW1, eb1):
    return pl.pallas_call(
        _knn_prep_body,
        grid=(NB, N // RB),
        in_specs=[
            pl.BlockSpec((1, RB, 3), lambda b, r: (b, r, 0)),
            pl.BlockSpec((1, 3, N), lambda b, r: (b, 0, 0)),
            pl.BlockSpec((1, RB, DIM), lambda b, r: (b, r, 0)),
            pl.BlockSpec((2 * DIM + 1, HID), lambda b, r: (0, 0)),
            pl.BlockSpec((HID,), lambda b, r: (0,)),
        ],
        out_specs=[
            pl.BlockSpec((1, RB, K), lambda b, r: (b, r, 0)),
            pl.BlockSpec((1, RB, HID), lambda b, r: (b, r, 0)),
            pl.BlockSpec((1, RB, TW), lambda b, r: (b, r, 0)),
        ],
        out_shape=[
            jax.ShapeDtypeStruct((NB, N, K), jnp.int32),
            jax.ShapeDtypeStruct((NB, N, HID), jnp.float32),
            jax.ShapeDtypeStruct((NB, N, TW), jnp.float32),
        ],
    )(coord, coordT, h, eW1, eb1)


def _sc_gather(table_flat, idx_flat):
    mesh = plsc.VectorSubcoreMesh(core_axis_name="c", subcore_axis_name="s")

    @functools.partial(
        pl.kernel,
        mesh=mesh,
        out_type=jax.ShapeDtypeStruct((EDGES, TW), jnp.float32),
        scratch_types=[
            pltpu.VMEM((CHUNK,), jnp.int32),
            pltpu.VMEM((CHUNK, TW), jnp.float32),
            pltpu.SemaphoreType.DMA,
        ],
    )
    def gk(table_hbm, idx_hbm, out_hbm, idx_v, rows_v, sem):
        wid = lax.axis_index("s") * SC_NC + lax.axis_index("c")
        base = wid * E_PER_W

        def body(c, _):
            off = base + c * CHUNK
            pltpu.sync_copy(idx_hbm.at[pl.ds(off, CHUNK)], idx_v)
            pltpu.async_copy(table_hbm.at[idx_v], rows_v, sem).wait()
            pltpu.sync_copy(rows_v, out_hbm.at[pl.ds(off, CHUNK)])
            return 0

        lax.fori_loop(0, NCHUNK, body, 0)

    return gk(table_flat, idx_flat)


def _edge_node_body(edge_ref, a_ref, h_ref, coord_ref,
                    eW1_ref, eW2_ref, eb2_ref, cW_ref, cb_ref,
                    nW1_ref, nb1_ref, nW2_ref, nb2_ref,
                    h_out_ref, coord_out_ref):
    E = RE * K
    ed = edge_ref[0]
    bj = ed[:, 0:HID]
    cj = ed[:, HID:HID + 3]
    ab = a_ref[0]
    cb = coord_ref[0]
    hb = h_ref[0]
    ai = jnp.broadcast_to(ab[:, None, :], (RE, K, HID)).reshape(E, HID)
    ci = jnp.broadcast_to(cb[:, None, :], (RE, K, 3)).reshape(E, 3)
    rel = ci - cj
    rd = jnp.sum(rel * rel, axis=1, keepdims=True)
    w1c = eW1_ref[2 * DIM:2 * DIM + 1, :]
    m1 = _silu(ai + bj + rd * w1c)
    m = _silu(jnp.dot(m1, eW2_ref[:], preferred_element_type=jnp.float32)
              + eb2_ref[:][None, :])
    w = jnp.dot(m, cW_ref[:], preferred_element_type=jnp.float32) + cb_ref[0]
    rel_n = rel / (jnp.sqrt(rd) + 1.0)
    contrib = (rel_n * w).reshape(RE, K, 3)
    coord_out_ref[0] = cb + jnp.sum(contrib, axis=1) * (1.0 / K)
    m_i = jnp.sum(m.reshape(RE, K, HID), axis=1)
    node_in = jnp.concatenate([hb, m_i], axis=1)
    t1 = _silu(jnp.dot(node_in, nW1_ref[:], preferred_element_type=jnp.float32)
               + nb1_ref[:][None, :])
    h_out_ref[0] = hb + (jnp.dot(t1, nW2_ref[:],
                                 preferred_element_type=jnp.float32)
                         + nb2_ref[:][None, :])


def _edge_node(edge, A, h, coord, lp):
    return pl.pallas_call(
        _edge_node_body,
        grid=(NB, N // RE),
        in_specs=[
            pl.BlockSpec((1, RE * K, TW), lambda b, r: (b, r, 0)),
            pl.BlockSpec((1, RE, HID), lambda b, r: (b, r, 0)),
            pl.BlockSpec((1, RE, DIM), lambda b, r: (b, r, 0)),
            pl.BlockSpec((1, RE, 3), lambda b, r: (b, r, 0)),
            pl.BlockSpec((2 * DIM + 1, HID), lambda b, r: (0, 0)),
            pl.BlockSpec((HID, HID), lambda b, r: (0, 0)),
            pl.BlockSpec((HID,), lambda b, r: (0,)),
            pl.BlockSpec((HID, 1), lambda b, r: (0, 0)),
            pl.BlockSpec((1,), lambda b, r: (0,)),
            pl.BlockSpec((DIM + HID, HID), lambda b, r: (0, 0)),
            pl.BlockSpec((HID,), lambda b, r: (0,)),
            pl.BlockSpec((HID, DIM), lambda b, r: (0, 0)),
            pl.BlockSpec((DIM,), lambda b, r: (0,)),
        ],
        out_specs=[
            pl.BlockSpec((1, RE, DIM), lambda b, r: (b, r, 0)),
            pl.BlockSpec((1, RE, 3), lambda b, r: (b, r, 0)),
        ],
        out_shape=[
            jax.ShapeDtypeStruct((NB, N, DIM), jnp.float32),
            jax.ShapeDtypeStruct((NB, N, 3), jnp.float32),
        ],
    )(edge, A, h, coord, lp["eW1"], lp["eW2"], lp["eb2"], lp["cW"], lp["cb"],
      lp["nW1"], lp["nb1"], lp["nW2"], lp["nb2"])


def _attn_body(h_ref, coord_ref, mu_ref, gamma_ref,
               Wq_ref, bq_ref, Wk_ref, bk_ref, Wv_ref, bv_ref,
               Wo_ref, bo_ref, aW_ref, ab_ref, cWf_ref, cbf_ref,
               out_ref):
    coord = coord_ref[0]
    hb = h_ref[0]
    cent = jnp.sum(coord, axis=0, keepdims=True) * (1.0 / N)
    dc = coord - cent
    dist = jnp.sqrt(jnp.sum(dc * dc, axis=1, keepdims=True))
    gamma = gamma_ref[0, 0]
    dev = dist - mu_ref[:][None, :]
    rbf = jnp.exp(-gamma * dev * dev)
    T = jnp.concatenate([rbf, hb], axis=1)
    q = jnp.dot(T, Wq_ref[:], preferred_element_type=jnp.float32) \
        + bq_ref[:][None, :]
    k = jnp.dot(T, Wk_ref[:], preferred_element_type=jnp.float32) \
        + bk_ref[:][None, :]
    cWf = cWf_ref[:]
    u = jnp.dot(Wo_ref[:], cWf, preferred_element_type=jnp.float32)
    wvu = jnp.dot(Wv_ref[:], u, preferred_element_type=jnp.float32)
    bvu = jnp.dot(bv_ref[:][None, :], u,
                  preferred_element_type=jnp.float32)[0, 0]
    vu = jnp.dot(T, wvu, preferred_element_type=jnp.float32) + bvu
    s1 = jnp.dot(bo_ref[:][None, :], cWf,
                 preferred_element_type=jnp.float32)[0, 0]
    aW = aW_ref[:]
    const = (cbf_ref[0] + ab_ref[0] * jnp.sum(cWf)
             + s1 * jnp.sum(aW))
    scale = 1.0 / np.sqrt(HID).astype(np.float32)
    acc = jnp.zeros((), jnp.float32)
    RS = 256
    for rb in range(N // RS):
        qb = q[rb * RS:(rb + 1) * RS, :]
        s = lax.dot_general(qb, k, (((1,), (1,)), ((), ())),
                            preferred_element_type=jnp.float32) * scale
        mx = jnp.max(s, axis=1, keepdims=True)
        p = jnp.exp(s - mx)
        sm = jnp.sum(p, axis=1, keepdims=True)
        r = jnp.dot(p, vu, preferred_element_type=jnp.float32) / sm
        acc = acc + jnp.sum(aW[rb * RS:(rb + 1) * RS, :] * r)
    out_ref[0] = jnp.full((8, 128), const + acc, jnp.float32)


def _attn(h, coord, params):
    ap = params["attn"]
    gamma = params["rbf"]["gamma"].reshape(1, 1)
    return pl.pallas_call(
        _attn_body,
        grid=(NB,),
        in_specs=[
            pl.BlockSpec((1, N, DIM), lambda b: (b, 0, 0)),
            pl.BlockSpec((1, N, 3), lambda b: (b, 0, 0)),
            pl.BlockSpec((BASIS,), lambda b: (0,)),
            pl.BlockSpec((1, 1), lambda b: (0, 0)),
            pl.BlockSpec((INNER, HID), lambda b: (0, 0)),
            pl.BlockSpec((HID,), lambda b: (0,)),
            pl.BlockSpec((INNER, HID), lambda b: (0, 0)),
            pl.BlockSpec((HID,), lambda b: (0,)),
            pl.BlockSpec((INNER, HID), lambda b: (0, 0)),
            pl.BlockSpec((HID,), lambda b: (0,)),
            pl.BlockSpec((HID, INNER), lambda b: (0, 0)),
            pl.BlockSpec((INNER,), lambda b: (0,)),
            pl.BlockSpec((N, 1), lambda b: (0, 0)),
            pl.BlockSpec((1,), lambda b: (0,)),
            pl.BlockSpec((INNER, 1), lambda b: (0, 0)),
            pl.BlockSpec((1,), lambda b: (0,)),
        ],
        out_specs=pl.BlockSpec((1, 8, 128), lambda b: (b, 0, 0)),
        out_shape=jax.ShapeDtypeStruct((NB, 8, 128), jnp.float32),
    )(h, coord, params["rbf"]["mu"], gamma,
      ap["Wq"], ap["bq"], ap["Wk"], ap["bk"], ap["Wv"], ap["bv"],
      ap["Wo"], ap["bo"], params["agg"]["W"], params["agg"]["b"],
      params["ch_agg"]["W"], params["ch_agg"]["b"])


def kernel(z, x, params):
    h, coord = z, x
    for lp in params["egnn"]:
        coordT = jnp.transpose(coord, (0, 2, 1))
        nbr, A, table = _knn_prep(coord, coordT, h, lp["eW1"], lp["eb1"])
        edge = _sc_gather(table.reshape(NB * N, TW), nbr.reshape(EDGES))
        h, coord = _edge_node(edge.reshape(NB, N * K, TW), A, h, coord, lp)
    return _attn(h, coord, params)[:, 0, 0][None, :]
